# baseline (speedup 1.0000x reference)
import jax
import jax.numpy as jnp
from jax.experimental import pallas as pl

_INPUT_LENGTH = 16384
_EMBED_DIM = 128
_REPEATS = 8
_TOTAL_LENGTH = 131072
_CHUNKS = 16
_CHUNK_ROWS = _INPUT_LENGTH // _CHUNKS


def _copy_body(x_ref, o_ref):
    o_ref[...] = x_ref[...]


def kernel(x):
    return pl.pallas_call(
        _copy_body,
        grid=(_CHUNKS, _REPEATS),
        in_specs=[pl.BlockSpec((_CHUNK_ROWS, _EMBED_DIM), lambda c, r: (c, 0))],
        out_specs=pl.BlockSpec(
            (_CHUNK_ROWS, _EMBED_DIM), lambda c, r: (r * _CHUNKS + c, 0)
        ),
        out_shape=jax.ShapeDtypeStruct((_TOTAL_LENGTH, _EMBED_DIM), jnp.float32),
    )(x)


# speedup vs baseline: 2.5681x; 2.5681x over previous
import jax
import jax.numpy as jnp
from jax.experimental import pallas as pl

_INPUT_LENGTH = 16384
_EMBED_DIM = 128
_TOTAL_LENGTH = 131072


def _copy_body(x_ref, o_ref):
    o_ref[: _INPUT_LENGTH] = x_ref[...]
    o_ref[_INPUT_LENGTH :] = x_ref[...]


def kernel(x):
    return pl.pallas_call(
        _copy_body,
        grid=(4,),
        in_specs=[pl.BlockSpec((_INPUT_LENGTH, _EMBED_DIM), lambda i: (0, 0))],
        out_specs=pl.BlockSpec((2 * _INPUT_LENGTH, _EMBED_DIM), lambda i: (i, 0)),
        out_shape=jax.ShapeDtypeStruct((_TOTAL_LENGTH, _EMBED_DIM), jnp.float32),
    )(x)
